# baseline (speedup 1.0000x reference)
import functools

import jax
import jax.numpy as jnp
from jax import lax
from jax.experimental import pallas as pl
from jax.experimental.pallas import tpu as pltpu
from jax.experimental.pallas import tpu_sc as plsc

N = 10000
E = 320000
D = 128
C = 64

NC = 2
NS = 16
NW = NC * NS

K = 128
CHUNKS = 80
CPP = 40
PER_TILE = CHUNKS * K
E_PAD = PER_TILE * NW

ACC_ROWS = 10240
ZR = 128
WB = ACC_ROWS // NS


def _mm_body(x_ref, wt_ref, o_ref):
    o_ref[...] = jnp.dot(x_ref[...], wt_ref[...],
                         preferred_element_type=jnp.float32)


def _tc_matmul(x, wt):
    n, _ = x.shape
    f_out = wt.shape[1]
    return pl.pallas_call(
        _mm_body,
        out_shape=jax.ShapeDtypeStruct((n, f_out), jnp.float32),
    )(x, wt)


def _combine_mm_body(a_ref, y_ref, p_ref, b_ref, wt_ref, o_ref):
    a = a_ref[0, 0]
    h = a * y_ref[...] + p_ref[:N, :] + p_ref[ACC_ROWS:ACC_ROWS + N, :] + b_ref[...]
    h = jnp.maximum(h, 0.0)
    o_ref[...] = jnp.dot(h, wt_ref[...], preferred_element_type=jnp.float32)


def _tc_combine_matmul(a, y, p, b, wt):
    f_in = y.shape[1]
    f_out = wt.shape[1]
    return pl.pallas_call(
        _combine_mm_body,
        in_specs=[
            pl.BlockSpec(memory_space=pltpu.SMEM),
            pl.BlockSpec(memory_space=pltpu.VMEM),
            pl.BlockSpec(memory_space=pltpu.VMEM),
            pl.BlockSpec(memory_space=pltpu.VMEM),
            pl.BlockSpec(memory_space=pltpu.VMEM),
        ],
        out_shape=jax.ShapeDtypeStruct((N, f_out), jnp.float32),
    )(jnp.reshape(a, (1, 1)), y, p, jnp.reshape(b, (1, f_in)), wt)


def _combine_relu_body(a_ref, y_ref, p_ref, b_ref, o_ref):
    a = a_ref[0, 0]
    h = a * y_ref[...] + p_ref[:N, :] + p_ref[ACC_ROWS:ACC_ROWS + N, :] + b_ref[...]
    o_ref[...] = jnp.maximum(h, 0.0)


def _tc_combine_relu(a, y, p, b):
    f = y.shape[1]
    return pl.pallas_call(
        _combine_relu_body,
        in_specs=[
            pl.BlockSpec(memory_space=pltpu.SMEM),
            pl.BlockSpec(memory_space=pltpu.VMEM),
            pl.BlockSpec(memory_space=pltpu.VMEM),
            pl.BlockSpec(memory_space=pltpu.VMEM),
        ],
        out_shape=jax.ShapeDtypeStruct((N, f), jnp.float32),
    )(jnp.reshape(a, (1, 1)), y, p, jnp.reshape(b, (1, f)))


def _final_mm_body(a_ref, h_ref, p_ref, wt_ref, b_ref, o_ref):
    a = a_ref[0, 0]
    g = a * h_ref[...] + p_ref[:N, :] + p_ref[ACC_ROWS:ACC_ROWS + N, :]
    o_ref[...] = jnp.dot(g, wt_ref[...],
                         preferred_element_type=jnp.float32) + b_ref[...]


def _tc_final_mm(a, h, p, wt, b):
    f_out = wt.shape[1]
    return pl.pallas_call(
        _final_mm_body,
        in_specs=[
            pl.BlockSpec(memory_space=pltpu.SMEM),
            pl.BlockSpec(memory_space=pltpu.VMEM),
            pl.BlockSpec(memory_space=pltpu.VMEM),
            pl.BlockSpec(memory_space=pltpu.VMEM),
            pl.BlockSpec(memory_space=pltpu.VMEM),
        ],
        out_shape=jax.ShapeDtypeStruct((N, f_out), jnp.float32),
    )(jnp.reshape(a, (1, 1)), h, p, wt, jnp.reshape(b, (1, f_out)))


def _spmm_sc(y, src, dst, f):
    mesh = plsc.VectorSubcoreMesh(core_axis_name="c", subcore_axis_name="s")

    @functools.partial(
        pl.kernel,
        mesh=mesh,
        out_type=jax.ShapeDtypeStruct((2 * ACC_ROWS, f), jnp.float32),
        scratch_types=[
            pltpu.VMEM_SHARED((ACC_ROWS, f), jnp.float32),
            pltpu.VMEM((K,), jnp.int32),
            pltpu.VMEM((K,), jnp.int32),
            pltpu.VMEM((K,), jnp.int32),
            pltpu.VMEM((K,), jnp.int32),
            pltpu.VMEM((K, f), jnp.float32),
            pltpu.VMEM((K, f), jnp.float32),
            pltpu.SemaphoreType.DMA,
            pltpu.SemaphoreType.DMA,
            pltpu.SemaphoreType.DMA,
            pltpu.SemaphoreType.DMA,
            pltpu.SemaphoreType.DMA,
        ],
    )
    def k(y_hbm, src_hbm, dst_hbm, out_hbm, acc, srcv0, dstv0, srcv1, dstv1,
          rows0, rows1, gsem0, gsem1, isem0, isem1, zsem):
        c = lax.axis_index("c")
        s = lax.axis_index("s")
        w = c * NS + s

        lanes = f // 16

        def zb(i, carry):
            r = i // lanes
            col = (i % lanes) * 16
            rows0[r, pl.ds(col, 16)] = jnp.zeros((16,), jnp.float32)
            return carry

        lax.fori_loop(0, K * lanes, zb, 0)

        zcopies = [
            pltpu.async_copy(
                rows0, acc.at[pl.ds(s * (ACC_ROWS // NS) + i * K, K)], zsem)
            for i in range(ACC_ROWS // NS // K)
        ]
        for zc in zcopies:
            zc.wait()
        plsc.subcore_barrier()

        base = w * PER_TILE

        def ixload(t, sv, dv, isem):
            off = base + t * K
            pltpu.async_copy(src_hbm.at[pl.ds(off, K)], sv, isem)
            pltpu.async_copy(dst_hbm.at[pl.ds(off, K)], dv, isem)

        def ixwait(sv, dv, isem):
            pltpu.make_async_copy(src_hbm.at[pl.ds(0, K)], sv, isem).wait()
            pltpu.make_async_copy(dst_hbm.at[pl.ds(0, K)], dv, isem).wait()

        def gather(sv, buf):
            pltpu.sync_copy(y_hbm.at[sv], buf)

        def scatter(buf, dv):
            pltpu.sync_copy(buf, acc.at[dv], add=True)

        ixload(0, srcv0, dstv0, isem0)
        ixload(1, srcv1, dstv1, isem1)

        def step(u, carry):
            t0 = 2 * u
            ixwait(srcv0, dstv0, isem0)
            gather(srcv0, rows0)
            scatter(rows0, dstv0)
            ixload(t0 + 2, srcv0, dstv0, isem0)
            ixwait(srcv1, dstv1, isem1)
            gather(srcv1, rows1)
            scatter(rows1, dstv1)
            ixload(t0 + 3, srcv1, dstv1, isem1)
            return carry

        lax.fori_loop(0, CHUNKS // 2 - 1, step, 0)
        ixwait(srcv0, dstv0, isem0)
        gather(srcv0, rows0)
        scatter(rows0, dstv0)
        ixwait(srcv1, dstv1, isem1)
        gather(srcv1, rows1)
        scatter(rows1, dstv1)
        plsc.subcore_barrier()

        pltpu.sync_copy(acc.at[pl.ds(s * WB, WB)],
                        out_hbm.at[pl.ds(c * ACC_ROWS + s * WB, WB)])

    return k(y, src, dst)


def kernel(x, edge_index, eps, W0, b0, W1, b1, W2, b2):
    dst = edge_index[0].astype(jnp.int32)
    src = edge_index[1].astype(jnp.int32)
    pad = E_PAD - E
    src_p = jnp.concatenate([src, jnp.zeros((pad,), jnp.int32)])
    dst_p = jnp.concatenate([dst, jnp.full((pad,), N, jnp.int32)])
    a = 1.0 + eps

    y0 = _tc_matmul(x, W0.T)
    s0 = _spmm_sc(y0, src_p, dst_p, D)
    y1 = _tc_combine_matmul(a[0], y0, s0, b0, W1.T)
    s1 = _spmm_sc(y1, src_p, dst_p, D)
    h2 = _tc_combine_relu(a[1], y1, s1, b1)
    s2 = _spmm_sc(h2, src_p, dst_p, D)
    z = _tc_final_mm(a[2], h2, s2, W2.T, b2)
    return z


# speedup vs baseline: 1.5119x; 1.5119x over previous
import functools

import jax
import jax.numpy as jnp
from jax import lax
from jax.experimental import pallas as pl
from jax.experimental.pallas import tpu as pltpu
from jax.experimental.pallas import tpu_sc as plsc

N = 10000
E = 320000
D = 128
C = 64
HF = 64

NC = 2
NS = 16
NW = NC * NS

K = 128
CHUNKS = 80
PER_TILE = CHUNKS * K
E_PAD = PER_TILE * NW

NPAD = 10240
ACC_ROWS = NPAD
WB = ACC_ROWS // NS


def _split(y):
    pad = jnp.zeros((NPAD - N, HF), jnp.float32)
    return jnp.stack([jnp.concatenate([y[:, :HF], pad], axis=0),
                      jnp.concatenate([y[:, HF:], pad], axis=0)])


def _mm_body(x_ref, wt_ref, o_ref, os_ref):
    y = jnp.dot(x_ref[...], wt_ref[...], preferred_element_type=jnp.float32)
    o_ref[...] = y
    os_ref[...] = _split(y)


def _tc_matmul(x, wt):
    n, _ = x.shape
    f_out = wt.shape[1]
    return pl.pallas_call(
        _mm_body,
        out_shape=(jax.ShapeDtypeStruct((n, f_out), jnp.float32),
                   jax.ShapeDtypeStruct((2, NPAD, HF), jnp.float32)),
    )(x, wt)


def _psum(p_ref, h):
    b0 = (2 * h) * ACC_ROWS
    b1 = (2 * h + 1) * ACC_ROWS
    return p_ref[b0:b0 + N, :] + p_ref[b1:b1 + N, :]


def _spmm_full(p_ref):
    return jnp.concatenate([_psum(p_ref, 0), _psum(p_ref, 1)], axis=1)


def _combine_mm_body(a_ref, y_ref, p_ref, b_ref, wt_ref, o_ref, os_ref):
    a = a_ref[0, 0]
    h = a * y_ref[...] + _spmm_full(p_ref) + b_ref[...]
    h = jnp.maximum(h, 0.0)
    y = jnp.dot(h, wt_ref[...], preferred_element_type=jnp.float32)
    o_ref[...] = y
    os_ref[...] = _split(y)


def _tc_combine_matmul(a, y, p, b, wt):
    f_in = y.shape[1]
    f_out = wt.shape[1]
    return pl.pallas_call(
        _combine_mm_body,
        in_specs=[
            pl.BlockSpec(memory_space=pltpu.SMEM),
            pl.BlockSpec(memory_space=pltpu.VMEM),
            pl.BlockSpec(memory_space=pltpu.VMEM),
            pl.BlockSpec(memory_space=pltpu.VMEM),
            pl.BlockSpec(memory_space=pltpu.VMEM),
        ],
        out_shape=(jax.ShapeDtypeStruct((N, f_out), jnp.float32),
                   jax.ShapeDtypeStruct((2, NPAD, HF), jnp.float32)),
    )(jnp.reshape(a, (1, 1)), y, p, jnp.reshape(b, (1, f_in)), wt)


def _combine_relu_body(a_ref, y_ref, p_ref, b_ref, o_ref, os_ref):
    a = a_ref[0, 0]
    h = a * y_ref[...] + _spmm_full(p_ref) + b_ref[...]
    h = jnp.maximum(h, 0.0)
    o_ref[...] = h
    os_ref[...] = _split(h)


def _tc_combine_relu(a, y, p, b):
    f = y.shape[1]
    return pl.pallas_call(
        _combine_relu_body,
        in_specs=[
            pl.BlockSpec(memory_space=pltpu.SMEM),
            pl.BlockSpec(memory_space=pltpu.VMEM),
            pl.BlockSpec(memory_space=pltpu.VMEM),
            pl.BlockSpec(memory_space=pltpu.VMEM),
        ],
        out_shape=(jax.ShapeDtypeStruct((N, f), jnp.float32),
                   jax.ShapeDtypeStruct((2, NPAD, HF), jnp.float32)),
    )(jnp.reshape(a, (1, 1)), y, p, jnp.reshape(b, (1, f)))


def _final_mm_body(a_ref, h_ref, p_ref, wt_ref, b_ref, o_ref):
    a = a_ref[0, 0]
    g = a * h_ref[...] + _spmm_full(p_ref)
    o_ref[...] = jnp.dot(g, wt_ref[...],
                         preferred_element_type=jnp.float32) + b_ref[...]


def _tc_final_mm(a, h, p, wt, b):
    f_out = wt.shape[1]
    return pl.pallas_call(
        _final_mm_body,
        in_specs=[
            pl.BlockSpec(memory_space=pltpu.SMEM),
            pl.BlockSpec(memory_space=pltpu.VMEM),
            pl.BlockSpec(memory_space=pltpu.VMEM),
            pl.BlockSpec(memory_space=pltpu.VMEM),
            pl.BlockSpec(memory_space=pltpu.VMEM),
        ],
        out_shape=jax.ShapeDtypeStruct((N, f_out), jnp.float32),
    )(jnp.reshape(a, (1, 1)), h, p, wt, jnp.reshape(b, (1, f_out)))


def _spmm_sc(ysp, src, dst):
    mesh = plsc.VectorSubcoreMesh(core_axis_name="c", subcore_axis_name="s")

    @functools.partial(
        pl.kernel,
        mesh=mesh,
        out_type=jax.ShapeDtypeStruct((4 * ACC_ROWS, HF), jnp.float32),
        scratch_types=[
            pltpu.VMEM_SHARED((NPAD, HF), jnp.float32),
            pltpu.VMEM_SHARED((ACC_ROWS, HF), jnp.float32),
            pltpu.VMEM((K,), jnp.int32),
            pltpu.VMEM((K,), jnp.int32),
            pltpu.VMEM((K, HF), jnp.float32),
            pltpu.VMEM((K, HF), jnp.float32),
        ],
    )
    def k(ysp_hbm, src_hbm, dst_hbm, out_hbm, tbl, acc, srcv, dstv, rows, zbuf):
        c = lax.axis_index("c")
        s = lax.axis_index("s")
        w = c * NS + s
        base = w * PER_TILE

        lanes = HF // 16

        def zb(i, carry):
            r = i // lanes
            col = (i % lanes) * 16
            zbuf[r, pl.ds(col, 16)] = jnp.zeros((16,), jnp.float32)
            return carry

        lax.fori_loop(0, K * lanes, zb, 0)

        for h in range(2):
            pltpu.sync_copy(ysp_hbm.at[h, pl.ds(s * WB, WB)],
                            tbl.at[pl.ds(s * WB, WB)])
            for i in range(WB // K):
                pltpu.sync_copy(zbuf, acc.at[pl.ds(s * WB + i * K, K)])
            plsc.subcore_barrier()

            def step(t, carry):
                off = base + t * K
                pltpu.sync_copy(src_hbm.at[pl.ds(off, K)], srcv)
                pltpu.sync_copy(dst_hbm.at[pl.ds(off, K)], dstv)
                pltpu.sync_copy(tbl.at[srcv], rows)
                pltpu.sync_copy(rows, acc.at[dstv], add=True)
                return carry

            lax.fori_loop(0, CHUNKS, step, 0)
            plsc.subcore_barrier()

            pltpu.sync_copy(
                acc.at[pl.ds(s * WB, WB)],
                out_hbm.at[pl.ds((2 * h + c) * ACC_ROWS + s * WB, WB)])

    return k(ysp, src, dst)


def kernel(x, edge_index, eps, W0, b0, W1, b1, W2, b2):
    dst = edge_index[0].astype(jnp.int32)
    src = edge_index[1].astype(jnp.int32)
    pad = E_PAD - E
    src_p = jnp.concatenate([src, jnp.zeros((pad,), jnp.int32)])
    dst_p = jnp.concatenate([dst, jnp.full((pad,), N, jnp.int32)])
    a = 1.0 + eps

    y0, ysp0 = _tc_matmul(x, W0.T)
    s0 = _spmm_sc(ysp0, src_p, dst_p)
    y1, ysp1 = _tc_combine_matmul(a[0], y0, s0, b0, W1.T)
    s1 = _spmm_sc(ysp1, src_p, dst_p)
    h2, hsp2 = _tc_combine_relu(a[1], y1, s1, b1)
    s2 = _spmm_sc(hsp2, src_p, dst_p)
    z = _tc_final_mm(a[2], h2, s2, W2.T, b2)
    return z
